# Initial kernel scaffold; baseline (speedup 1.0000x reference)
import jax
import jax.numpy as jnp
from jax.experimental import pallas as pl


def kernel(x, nodes, batch_rows, batch_cols, full_rows, full_cols):
    raise NotImplementedError("write your pallas kernel here")


# speedup vs baseline: 5.7300x; 5.7300x over previous
import jax
import jax.numpy as jnp
from jax import lax
from jax.experimental import pallas as pl
from jax.experimental.pallas import tpu as pltpu
from jax.experimental.pallas import tpu_sc as plsc

N = 10000
B = 2048
DEG = 32
D = 128

NS = 16
L = 16
SLB = 48
SLF = DEG
RB = B // NS
RF = N // NS
TAB = 10240
CH = TAB // NS


def _rsqrt16(v):
    i = plsc.bitcast(v, jnp.int32)
    i = jnp.full((L,), 0x5F3759DF, jnp.int32) - lax.shift_right_logical(
        i, jnp.full((L,), 1, jnp.int32))
    y = plsc.bitcast(i, jnp.float32)
    for _ in range(3):
        y = y * (1.5 - 0.5 * v * y * y)
    return y


def _sc_body(x_hbm, cb_hbm, cf_hbm, tf_hbm, tfn_hbm, mr_hbm,
             cols_v, flg_v, tmp_v, csum_v,
             gbuf, obuf, mbuf, t_a, acc_v, sh_all, sh_tot, semA, semB):
    c = lax.axis_index("c")
    s = lax.axis_index("s")
    is_batch = c == 0

    ones16 = jnp.ones((L,), jnp.float32)
    zeros16 = jnp.zeros((L,), jnp.float32)

    def init_body(j, _):
        off = j * L
        tmp_v[pl.ds(off, L)] = jnp.full((L,), -1, jnp.int32)
        csum_v[pl.ds(off, L)] = zeros16
        mbuf[pl.ds(off, L)] = zeros16
        return 0
    lax.fori_loop(0, TAB // L, init_body, 0)

    def phase1(R, SL):
        nq = SL // L

        def body(r, _):
            base = r * SL
            vqs = [cols_v[pl.ds(base + q * L, L)] for q in range(nq)]
            ids = [lax.iota(jnp.int32, L) + (base + q * L + 1)
                   for q in range(nq)]
            for q in range(nq):
                plsc.store_scatter(tmp_v, [vqs[q]], ids[q])
            for q in range(nq):
                g = plsc.load_gather(tmp_v, [vqs[q]])
                fq = g == ids[q]
                flg_v[pl.ds(base + q * L, L)] = jnp.where(fq, 1.0, 0.0)
                plsc.addupdate_scatter(csum_v, [vqs[q]], ones16, mask=fq)
            return 0
        lax.fori_loop(0, R, body, 0)

    @pl.when(is_batch)
    def _():
        pltpu.sync_copy(cb_hbm.at[pl.ds(s * RB * SLB, RB * SLB)],
                        cols_v.at[pl.ds(0, RB * SLB)])
        phase1(RB, SLB)

    @pl.when(jnp.logical_not(is_batch))
    def _():
        pltpu.sync_copy(cf_hbm.at[pl.ds(s * RF * SLF, RF * SLF)],
                        cols_v.at[pl.ds(0, RF * SLF)])
        phase1(RF, SLF)

    pltpu.sync_copy(csum_v, sh_all.at[s])
    plsc.subcore_barrier()
    off = s * CH

    def zacc(j, _):
        acc_v[pl.ds(j * L, L)] = zeros16
        return 0
    lax.fori_loop(0, CH // L, zacc, 0)
    for t in range(NS):
        pltpu.sync_copy(sh_all.at[t, pl.ds(off, CH)], t_a)

        def addt(j, _):
            o = j * L
            acc_v[pl.ds(o, L)] = acc_v[pl.ds(o, L)] + t_a[pl.ds(o, L)]
            return 0
        lax.fori_loop(0, CH // L, addt, 0)
    pltpu.sync_copy(acc_v, sh_tot.at[pl.ds(off, CH)])
    plsc.subcore_barrier()
    pltpu.sync_copy(sh_tot, csum_v)

    def phase2(R, SL, out_hbm, base_row, do_mask, G):
        nq = SL // L
        GSL = G * SL
        NG = R // G

        def start(g, slot, dsem):
            pltpu.async_copy(
                x_hbm.at[cols_v.at[pl.ds(g * GSL, GSL)]],
                gbuf.at[pl.ds(slot * GSL, GSL)], dsem)

        def wait(slot, dsem):
            pltpu.make_async_copy(
                x_hbm.at[pl.ds(0, GSL)],
                gbuf.at[pl.ds(slot * GSL, GSL)], dsem).wait()

        def compute_row(r, slot_base, j):
            base = r * SL
            ftot = zeros16
            wqs = []
            for q in range(nq):
                vq = cols_v[pl.ds(base + q * L, L)]
                cnt = plsc.load_gather(csum_v, [vq])
                ff = flg_v[pl.ds(base + q * L, L)]
                wqs.append(ff * _rsqrt16(jnp.maximum(cnt, 1.0)))
                ftot = ftot + ff
            rowsum = jnp.sum(ftot)
            rsr = _rsqrt16(jnp.full((L,), rowsum))
            accs = [zeros16] * (D // L)
            for k in range(SL):
                wk = jnp.full((L,), wqs[k // L][k % L])
                for h in range(D // L):
                    accs[h] = accs[h] + wk * gbuf[slot_base + j * SL + k,
                                                  pl.ds(h * L, L)]
            for h in range(D // L):
                obuf[pl.ds(j * D + h * L, L)] = accs[h] * rsr
            if do_mask:
                val = ones16 / jnp.full((L,), rowsum)
                for q in range(nq):
                    vq = cols_v[pl.ds(base + q * L, L)]
                    fq = flg_v[pl.ds(base + q * L, L)] > 0.5
                    plsc.store_scatter(mbuf, [vq], val, mask=fq)
                pltpu.sync_copy(mbuf.at[pl.ds(0, N)],
                                mr_hbm.at[pl.ds((base_row + r) * N, N)])
                for q in range(nq):
                    vq = cols_v[pl.ds(base + q * L, L)]
                    fq = flg_v[pl.ds(base + q * L, L)] > 0.5
                    plsc.store_scatter(mbuf, [vq], zeros16, mask=fq)

        def compute_group(g, slot):
            for j in range(G):
                compute_row(g * G + j, slot * GSL, j)
            pltpu.sync_copy(obuf.at[pl.ds(0, G * D)],
                            out_hbm.at[pl.ds((base_row + g * G) * D, G * D)])

        def pair_body(gg, _):
            g0 = gg * 2
            start(g0 + 1, 1, semB)
            wait(0, semA)
            compute_group(g0, 0)
            start(jnp.minimum(g0 + 2, NG - 1), 0, semA)
            wait(1, semB)
            compute_group(g0 + 1, 1)
            return 0

        start(0, 0, semA)
        lax.fori_loop(0, NG // 2, pair_body, 0)
        wait(0, semA)
        if R - NG * G:
            r = R - 1
            pltpu.async_copy(
                x_hbm.at[cols_v.at[pl.ds(r * SL, SL)]],
                gbuf.at[pl.ds(0, SL)], semA)
            pltpu.make_async_copy(
                x_hbm.at[pl.ds(0, SL)], gbuf.at[pl.ds(0, SL)], semA).wait()
            compute_row(r, 0, 0)
            pltpu.sync_copy(obuf.at[pl.ds(0, D)],
                            out_hbm.at[pl.ds((base_row + r) * D, D)])

    @pl.when(is_batch)
    def _():
        phase2(RB, SLB, tf_hbm, s * RB, True, 2)

    @pl.when(jnp.logical_not(is_batch))
    def _():
        phase2(RF, SLF, tfn_hbm, s * RF, False, 4)


@jax.jit
def _run(x, cb, cf):
    mesh = plsc.VectorSubcoreMesh(core_axis_name="c", subcore_axis_name="s")
    f = pl.kernel(
        _sc_body,
        out_type=(
            jax.ShapeDtypeStruct((B * D,), jnp.float32),
            jax.ShapeDtypeStruct((N * D,), jnp.float32),
            jax.ShapeDtypeStruct((B * N,), jnp.float32),
        ),
        mesh=mesh,
        compiler_params=pltpu.CompilerParams(needs_layout_passes=False),
        scratch_types=[
            pltpu.VMEM((RF * SLF,), jnp.int32),
            pltpu.VMEM((RF * SLF,), jnp.float32),
            pltpu.VMEM((TAB,), jnp.int32),
            pltpu.VMEM((TAB,), jnp.float32),
            pltpu.VMEM((2 * 4 * SLF, D), jnp.float32),
            pltpu.VMEM((4 * D,), jnp.float32),
            pltpu.VMEM((TAB,), jnp.float32),
            pltpu.VMEM((CH,), jnp.float32),
            pltpu.VMEM((CH,), jnp.float32),
            pltpu.VMEM_SHARED((NS, TAB), jnp.float32),
            pltpu.VMEM_SHARED((TAB,), jnp.float32),
            pltpu.SemaphoreType.DMA,
            pltpu.SemaphoreType.DMA,
        ],
    )
    return f(x, cb, cf)


def kernel(x, nodes, batch_rows, batch_cols, full_rows, full_cols):
    cb = jnp.concatenate(
        [batch_cols.reshape(B, DEG),
         jnp.broadcast_to(nodes[:, None], (B, SLB - DEG))],
        axis=1).reshape(-1)
    cf = full_cols.reshape(-1)
    tf, tfn, mr = _run(x, cb, cf)
    return (tf.reshape(B, D), tfn.reshape(N, D), mr.reshape(B, N))
